# baseline (speedup 1.0000x reference)
import functools

import jax
import jax.numpy as jnp
from jax import lax
from jax.experimental import pallas as pl
from jax.experimental.pallas import tpu as pltpu
from jax.experimental.pallas import tpu_sc as plsc

NUM_FIELDS = 26
EMBEDDING_DIM = 32
LANES = 16
ROW_W = 128
DMAS_PER_CHUNK = 8
TILE_R = 128


def _make_sc_transpose(n_rows):
    info = plsc.get_sparse_core_info()
    nc, ns = info.num_cores, info.num_subcores
    nw = nc * ns
    n_full = n_rows // TILE_R
    per_w = -(-n_full // nw)
    last_w = n_full - per_w * (nw - 1)
    blk_words = TILE_R * EMBEDDING_DIM
    n_rows_pad = -(-n_rows // TILE_R) * TILE_R
    out_words = n_rows_pad * EMBEDDING_DIM
    tail_rows = n_rows - n_full * TILE_R
    tail_words = tail_rows * EMBEDDING_DIM
    mesh = plsc.VectorSubcoreMesh(core_axis_name="c", subcore_axis_name="s")

    @functools.partial(
        pl.kernel,
        mesh=mesh,
        compiler_params=pltpu.CompilerParams(
            use_tc_tiling_on_sc=True, needs_layout_passes=False),
        out_type=jax.ShapeDtypeStruct((out_words,), jnp.float32),
        scratch_types=[
            pltpu.VMEM((2, EMBEDDING_DIM, TILE_R), jnp.float32),
            pltpu.VMEM((2, blk_words), jnp.float32),
            pltpu.VMEM((tail_words,), jnp.float32),
            pltpu.SemaphoreType.DMA,
            pltpu.SemaphoreType.DMA,
        ],
    )
    def sc_transpose(table_t, tail_hbm, out_hbm, stag, outb, tstag, lsem, ssem):
        wid = lax.axis_index("s") * nc + lax.axis_index("c")
        j0 = wid * per_w
        nt = jnp.where(wid == nw - 1, last_w, per_w)
        d_rows = [lax.iota(jnp.int32, LANES) + d0 for d0 in range(0, EMBEDDING_DIM, LANES)]

        def load(t, buf):
            return pltpu.async_copy(
                table_t.at[pl.ds(0, EMBEDDING_DIM), pl.ds((j0 + t) * TILE_R, TILE_R)],
                stag.at[buf], lsem)

        load(0, 0)

        def body(t, carry):
            buf = lax.rem(t, 2)
            pltpu.make_async_copy(
                table_t.at[pl.ds(0, EMBEDDING_DIM), pl.ds((j0 + t) * TILE_R, TILE_R)],
                stag.at[buf], lsem).wait()

            @pl.when(t + 1 < nt)
            def _():
                load(t + 1, 1 - buf)

            @pl.when(t >= 2)
            def _():
                pltpu.make_async_copy(
                    outb.at[buf], out_hbm.at[pl.ds(0, blk_words)], ssem).wait()

            sbuf = stag.at[buf]
            for l in range(TILE_R):
                col = jnp.full((LANES,), l, jnp.int32)
                for k, d0 in enumerate(range(0, EMBEDDING_DIM, LANES)):
                    v = plsc.load_gather(sbuf, [d_rows[k], col])
                    outb[buf, pl.ds(l * EMBEDDING_DIM + d0, LANES)] = v
            pltpu.async_copy(
                outb.at[buf], out_hbm.at[pl.ds((j0 + t) * blk_words, blk_words)], ssem)
            return carry

        lax.fori_loop(0, nt, body, 0)
        for _ in range(2):
            pltpu.make_async_copy(
                outb.at[0], out_hbm.at[pl.ds(0, blk_words)], ssem).wait()

        @pl.when(wid == 0)
        def _():
            pltpu.sync_copy(tail_hbm, tstag)
            pltpu.sync_copy(tstag, out_hbm.at[pl.ds(n_full * blk_words, tail_words)])

    return sc_transpose


def _make_sc_gather(n_flat, dim):
    info = plsc.get_sparse_core_info()
    nc, ns = info.num_cores, info.num_subcores
    nw = nc * ns
    per_w = n_flat // nw
    assert per_w * nw == n_flat and per_w % (ROW_W * DMAS_PER_CHUNK) == 0
    n_idx_rows = per_w // ROW_W
    chunk = ROW_W * DMAS_PER_CHUNK
    n_chunks = per_w // chunk

    mesh = plsc.VectorSubcoreMesh(core_axis_name="c", subcore_axis_name="s")

    @functools.partial(
        pl.kernel,
        mesh=mesh,
        compiler_params=pltpu.CompilerParams(use_tc_tiling_on_sc=False),
        out_type=jax.ShapeDtypeStruct((n_flat, dim), jnp.float32),
        scratch_types=[
            pltpu.VMEM((n_idx_rows, ROW_W), jnp.int32),
            pltpu.VMEM((n_idx_rows, ROW_W), jnp.int32),
            pltpu.VMEM((chunk, dim), jnp.float32),
            pltpu.SemaphoreType.DMA,
        ],
    )
    def sc_gather(idx_hbm, off_hbm, table_hbm, out_hbm, idx_v, off_v, rows_v, gsem):
        wid = lax.axis_index("s") * nc + lax.axis_index("c")
        base_row = wid * n_idx_rows
        pltpu.sync_copy(idx_hbm.at[pl.ds(base_row, n_idx_rows)], idx_v)
        pltpu.sync_copy(off_hbm, off_v)

        def add_body(r, carry):
            for k in range(ROW_W // LANES):
                sl = pl.ds(k * LANES, LANES)
                idx_v[r, sl] = idx_v[r, sl] + off_v[r, sl]
            return carry

        lax.fori_loop(0, n_idx_rows, add_body, 0)

        def chunk_body(t, carry):
            copies = []
            for b in range(DMAS_PER_CHUNK):
                copies.append(pltpu.async_copy(
                    table_hbm.at[idx_v.at[t * DMAS_PER_CHUNK + b]],
                    rows_v.at[pl.ds(b * ROW_W, ROW_W)],
                    gsem,
                ))
            for c in copies:
                c.wait()
            pltpu.sync_copy(
                rows_v,
                out_hbm.at[pl.ds((base_row + t * DMAS_PER_CHUNK) * ROW_W, chunk)],
            )
            return carry

        lax.fori_loop(0, n_chunks, chunk_body, 0)

    return sc_gather


def kernel(categorical_inputs, embedding_weight, offsets):
    batch, n_fields = categorical_inputs.shape
    n_flat = batch * n_fields
    idx_flat = categorical_inputs.astype(jnp.int32).reshape(n_flat // ROW_W, ROW_W)
    info = plsc.get_sparse_core_info()
    per_w = n_flat // (info.num_cores * info.num_subcores)
    off_pattern = jnp.tile(
        offsets[:n_fields].astype(jnp.int32), per_w // n_fields
    ).reshape(per_w // ROW_W, ROW_W)

    n_rows = embedding_weight.shape[0]
    n_full = n_rows // TILE_R
    tail = jax.lax.slice(
        embedding_weight, (n_full * TILE_R, 0), (n_rows, EMBEDDING_DIM)
    ).reshape(-1)
    sc_transpose = _make_sc_transpose(n_rows)
    scratch = sc_transpose(embedding_weight.T, tail)
    table_lin = scratch.reshape(-1, EMBEDDING_DIM)

    sc_gather = _make_sc_gather(n_flat, EMBEDDING_DIM)
    out_flat = sc_gather(idx_flat, off_pattern, table_lin)
    return out_flat.reshape(batch, n_fields, EMBEDDING_DIM)


# speedup vs baseline: 1.2663x; 1.2663x over previous
import functools

import jax
import jax.numpy as jnp
from jax import lax
from jax.experimental import pallas as pl
from jax.experimental.pallas import tpu as pltpu
from jax.experimental.pallas import tpu_sc as plsc

NUM_FIELDS = 26
EMBEDDING_DIM = 32
LANES = 16
ROW_W = 128
DMAS_PER_CHUNK = 8
TILE_R = 128


def _make_sc_transpose(n_rows):
    info = plsc.get_sparse_core_info()
    nc, ns = info.num_cores, info.num_subcores
    nw = nc * ns
    n_full = n_rows // TILE_R
    npairs = n_full // 2
    per_w = -(-npairs // nw)
    last_w = npairs - per_w * (nw - 1)
    blk_words = TILE_R * EMBEDDING_DIM
    n_rows_pad = -(-n_rows // TILE_R) * TILE_R
    out_words = n_rows_pad * EMBEDDING_DIM
    tail_rows = n_rows - n_full * TILE_R
    tail_words = tail_rows * EMBEDDING_DIM
    NBUF = 3
    mesh = plsc.VectorSubcoreMesh(core_axis_name="c", subcore_axis_name="s")

    @functools.partial(
        pl.kernel,
        mesh=mesh,
        compiler_params=pltpu.CompilerParams(
            use_tc_tiling_on_sc=True, needs_layout_passes=False),
        out_type=jax.ShapeDtypeStruct((out_words,), jnp.float32),
        scratch_types=(
            [pltpu.VMEM((NBUF, 2, EMBEDDING_DIM, TILE_R), jnp.float32)]
            + [pltpu.VMEM((2 * blk_words,), jnp.float32) for _ in range(NBUF)]
            + [pltpu.VMEM((tail_words,), jnp.float32),
               pltpu.SemaphoreType.DMA,
               pltpu.SemaphoreType.DMA]
        ),
    )
    def sc_transpose(table_t, tail_hbm, out_hbm, stag, outb0, outb1, outb2,
                     tstag, lsem, ssem):
        outbs = [outb0, outb1, outb2]
        wid = lax.axis_index("s") * nc + lax.axis_index("c")
        j0 = wid * per_w * 2
        nt = jnp.where(wid == nw - 1, last_w, per_w)
        n_outer = -(-per_w // NBUF)
        scat_base = [
            (lax.iota(jnp.int32, LANES) + l0) * EMBEDDING_DIM
            for l0 in range(0, TILE_R, LANES)
        ]

        def load(t, buf, c):
            return pltpu.async_copy(
                table_t.at[pl.ds(0, EMBEDDING_DIM),
                           pl.ds((j0 + 2 * t + c) * TILE_R, TILE_R)],
                stag.at[buf, c], lsem)

        for p in range(NBUF - 1):
            @pl.when(p < nt)
            def _(p=p):
                load(p, p, 0)
                load(p, p, 1)

        def outer(g, carry):
            for b in range(NBUF):
                t = g * NBUF + b

                @pl.when(t < nt)
                def _(t=t, b=b):
                    for c in range(2):
                        pltpu.make_async_copy(
                            table_t.at[pl.ds(0, EMBEDDING_DIM), pl.ds(0, TILE_R)],
                            stag.at[b, c], lsem).wait()

                    @pl.when(t + NBUF - 1 < nt)
                    def _():
                        load(t + NBUF - 1, (b + NBUF - 1) % NBUF, 0)
                        load(t + NBUF - 1, (b + NBUF - 1) % NBUF, 1)

                    @pl.when(t >= NBUF - 1)
                    def _():
                        pltpu.make_async_copy(
                            outbs[0], out_hbm.at[pl.ds(0, 2 * blk_words)],
                            ssem).wait()

                    ob = outbs[b]
                    for c in range(2):
                        sbuf = stag.at[b, c]
                        for d in range(EMBEDDING_DIM):
                            vs = [sbuf[d, pl.ds(l0, LANES)]
                                  for l0 in range(0, TILE_R, LANES)]
                            for m in range(TILE_R // LANES):
                                plsc.store_scatter(
                                    ob, [scat_base[m] + (c * blk_words + d)],
                                    vs[m])
                    pltpu.async_copy(
                        ob,
                        out_hbm.at[pl.ds((j0 + 2 * t) * blk_words,
                                         2 * blk_words)], ssem)
            return carry

        lax.fori_loop(0, n_outer, outer, 0)
        for p in range(NBUF - 1):
            pltpu.make_async_copy(
                outbs[0], out_hbm.at[pl.ds(0, 2 * blk_words)], ssem).wait()

        @pl.when(wid == 0)
        def _():
            pltpu.sync_copy(tail_hbm, tstag)
            pltpu.sync_copy(tstag, out_hbm.at[pl.ds(n_full * blk_words, tail_words)])

    return sc_transpose


def _make_sc_gather(n_flat, dim):
    info = plsc.get_sparse_core_info()
    nc, ns = info.num_cores, info.num_subcores
    nw = nc * ns
    per_w = n_flat // nw
    assert per_w * nw == n_flat and per_w % (ROW_W * DMAS_PER_CHUNK) == 0
    n_idx_rows = per_w // ROW_W
    chunk = ROW_W * DMAS_PER_CHUNK
    n_chunks = per_w // chunk

    mesh = plsc.VectorSubcoreMesh(core_axis_name="c", subcore_axis_name="s")

    @functools.partial(
        pl.kernel,
        mesh=mesh,
        compiler_params=pltpu.CompilerParams(use_tc_tiling_on_sc=False),
        out_type=jax.ShapeDtypeStruct((n_flat, dim), jnp.float32),
        scratch_types=[
            pltpu.VMEM((n_idx_rows, ROW_W), jnp.int32),
            pltpu.VMEM((n_idx_rows, ROW_W), jnp.int32),
            pltpu.VMEM((chunk, dim), jnp.float32),
            pltpu.SemaphoreType.DMA,
        ],
    )
    def sc_gather(idx_hbm, off_hbm, table_hbm, out_hbm, idx_v, off_v, rows_v, gsem):
        wid = lax.axis_index("s") * nc + lax.axis_index("c")
        base_row = wid * n_idx_rows
        pltpu.sync_copy(idx_hbm.at[pl.ds(base_row, n_idx_rows)], idx_v)
        pltpu.sync_copy(off_hbm, off_v)

        def add_body(r, carry):
            for k in range(ROW_W // LANES):
                sl = pl.ds(k * LANES, LANES)
                idx_v[r, sl] = idx_v[r, sl] + off_v[r, sl]
            return carry

        lax.fori_loop(0, n_idx_rows, add_body, 0)

        def chunk_body(t, carry):
            copies = []
            for b in range(DMAS_PER_CHUNK):
                copies.append(pltpu.async_copy(
                    table_hbm.at[idx_v.at[t * DMAS_PER_CHUNK + b]],
                    rows_v.at[pl.ds(b * ROW_W, ROW_W)],
                    gsem,
                ))
            for c in copies:
                c.wait()
            pltpu.sync_copy(
                rows_v,
                out_hbm.at[pl.ds((base_row + t * DMAS_PER_CHUNK) * ROW_W, chunk)],
            )
            return carry

        lax.fori_loop(0, n_chunks, chunk_body, 0)

    return sc_gather


def kernel(categorical_inputs, embedding_weight, offsets):
    batch, n_fields = categorical_inputs.shape
    n_flat = batch * n_fields
    idx_flat = categorical_inputs.astype(jnp.int32).reshape(n_flat // ROW_W, ROW_W)
    info = plsc.get_sparse_core_info()
    per_w = n_flat // (info.num_cores * info.num_subcores)
    off_pattern = jnp.tile(
        offsets[:n_fields].astype(jnp.int32), per_w // n_fields
    ).reshape(per_w // ROW_W, ROW_W)

    n_rows = embedding_weight.shape[0]
    n_full = n_rows // TILE_R
    tail = jax.lax.slice(
        embedding_weight, (n_full * TILE_R, 0), (n_rows, EMBEDDING_DIM)
    ).reshape(-1)
    sc_transpose = _make_sc_transpose(n_rows)
    scratch = sc_transpose(embedding_weight.T, tail)
    table_lin = scratch.reshape(-1, EMBEDDING_DIM)

    sc_gather = _make_sc_gather(n_flat, EMBEDDING_DIM)
    out_flat = sc_gather(idx_flat, off_pattern, table_lin)
    return out_flat.reshape(batch, n_fields, EMBEDDING_DIM)
